# baseline (speedup 1.0000x reference)
import functools

import jax
import jax.numpy as jnp
from jax.experimental import pallas as pl
from jax.experimental.pallas import tpu as pltpu

NBLK = 32
KC = 1024


def _rvq_kernel(x_ref, cb_ref, cn_ref, qout_ref, idx_ref, loss_ref, r_ref, *,
                nq, ksize):
    l = pl.program_id(1)

    @pl.when(l == 0)
    def _init():
        r_ref[...] = x_ref[...]

    r = r_ref[...]
    nb, d = r.shape
    cb = cb_ref[0]
    hi = jax.lax.Precision.HIGHEST

    fn = jnp.sum(r * r, axis=1, keepdims=True)
    rbf = r.astype(jnp.bfloat16)
    nchunks = ksize // KC
    half = nchunks // 2
    half_v = []
    half_i = []
    best_val = jnp.full((nb, 1), -jnp.inf, jnp.float32)
    best_idx = jnp.zeros((nb, 1), jnp.int32)
    for kb in range(nchunks):
        cchunk = cb[kb * KC:(kb + 1) * KC, :]
        cn = cn_ref[0, :, kb * KC:(kb + 1) * KC]
        cbf = cchunk.astype(jnp.bfloat16)
        mm = jax.lax.dot_general(
            rbf, cbf, (((1,), (1,)), ((), ())),
            preferred_element_type=jnp.float32)
        u = -((fn - 2.0 * mm) + cn)
        m = jnp.max(u, axis=1, keepdims=True)
        a = jnp.argmax(u, axis=1).astype(jnp.int32)[:, None] + kb * KC
        upd = m > best_val
        best_val = jnp.where(upd, m, best_val)
        best_idx = jnp.where(upd, a, best_idx)
        if kb == half - 1 or kb == nchunks - 1:
            half_v.append(best_val)
            half_i.append(best_idx)
            best_val = jnp.full((nb, 1), -jnp.inf, jnp.float32)
            best_idx = jnp.zeros((nb, 1), jnp.int32)
    hb = jax.lax.bitcast_convert_type(half_v[0], jnp.uint32)
    hb = (hb + jnp.uint32(0x7FFF) + ((hb >> 16) & jnp.uint32(1))) \
        & jnp.uint32(0xFFFF0000)
    w0_bf = jax.lax.bitcast_convert_type(hb, jnp.float32)
    w1_wins = half_v[1] > w0_bf
    best_idx = jnp.where(w1_wins, half_i[1], half_i[0])

    q = jnp.zeros((nb, d), jnp.float32)
    for kb in range(ksize // KC):
        cchunk = cb[kb * KC:(kb + 1) * KC, :]
        lane = jax.lax.broadcasted_iota(jnp.int32, (nb, KC), 1) + kb * KC
        oh = (best_idx == lane).astype(jnp.float32)
        q = q + jax.lax.dot_general(
            oh, cchunk, (((1,), (0,)), ((), ())), precision=hi)

    rn = r - q
    r_ref[...] = rn
    idx_ref[...] = best_idx.reshape(1, nb, 1)
    loss_ref[...] = jnp.full((1, 1, 128), jnp.sum(rn * rn), jnp.float32)
    qout_ref[...] = x_ref[...] - rn


@functools.partial(jax.jit, static_argnames=())
def kernel(x, codebooks):
    b, t, d = x.shape
    nq, ksize, _ = codebooks.shape
    n = b * t
    nb = n // NBLK
    x2 = x.reshape(n, d)

    grid = (NBLK, nq)
    qout, idx3, lossp = pl.pallas_call(
        functools.partial(_rvq_kernel, nq=nq, ksize=ksize),
        grid=grid,
        in_specs=[
            pl.BlockSpec((nb, d), lambda i, l: (i, 0)),
            pl.BlockSpec((1, ksize, d), lambda i, l: (l, 0, 0)),
            pl.BlockSpec((1, 1, ksize), lambda i, l: (l, 0, 0)),
        ],
        out_specs=[
            pl.BlockSpec((nb, d), lambda i, l: (i, 0)),
            pl.BlockSpec((1, nb, 1), lambda i, l: (l * NBLK + i, 0, 0)),
            pl.BlockSpec((1, 1, 128), lambda i, l: (l * NBLK + i, 0, 0)),
        ],
        out_shape=[
            jax.ShapeDtypeStruct((n, d), jnp.float32),
            jax.ShapeDtypeStruct((nq * NBLK, nb, 1), jnp.int32),
            jax.ShapeDtypeStruct((nq * NBLK, 1, 128), jnp.float32),
        ],
        scratch_shapes=[pltpu.VMEM((nb, d), jnp.float32)],
        compiler_params=pltpu.CompilerParams(
            dimension_semantics=("arbitrary", "arbitrary"),
        ),
    )(x2, codebooks, jnp.sum(codebooks**2, axis=2)[:, None, :])

    indices = idx3.reshape(nq, b, t)
    losses = 1.25 * lossp[:, 0, 0].reshape(nq, NBLK).sum(axis=1) / (n * d)
    return qout.reshape(b, t, d), indices, losses


# speedup vs baseline: 1.7151x; 1.7151x over previous
import functools

import jax
import jax.numpy as jnp
from jax.experimental import pallas as pl
from jax.experimental.pallas import tpu as pltpu

NBLK = 32
KC = 1024


def _rvq_kernel(x_ref, cb_ref, cn_ref, qout_ref, idx_ref, loss_ref, r_ref, *,
                nq, ksize):
    l = pl.program_id(1)

    @pl.when(l == 0)
    def _init():
        r_ref[...] = x_ref[...]

    r = r_ref[...]
    nb, d = r.shape
    cb = cb_ref[0]
    hi = jax.lax.Precision.HIGHEST

    fn = jnp.sum(r * r, axis=1, keepdims=True)
    rbf = r.astype(jnp.bfloat16)
    nchunks = ksize // KC
    half = nchunks // 2
    half_v = []
    half_i = []
    best_val = jnp.full((nb, 1), -jnp.inf, jnp.float32)
    best_idx = jnp.zeros((nb, 1), jnp.int32)
    for kb in range(nchunks):
        cchunk = cb[kb * KC:(kb + 1) * KC, :]
        cn = cn_ref[0, :, kb * KC:(kb + 1) * KC]
        cbf = cchunk.astype(jnp.bfloat16)
        mm = jax.lax.dot_general(
            rbf, cbf, (((1,), (1,)), ((), ())),
            preferred_element_type=jnp.float32)
        u = -((fn - 2.0 * mm) + cn)
        m = jnp.max(u, axis=1, keepdims=True)
        a = jnp.argmax(u, axis=1).astype(jnp.int32)[:, None] + kb * KC
        upd = m > best_val
        best_val = jnp.where(upd, m, best_val)
        best_idx = jnp.where(upd, a, best_idx)
        if kb == half - 1 or kb == nchunks - 1:
            half_v.append(best_val)
            half_i.append(best_idx)
            best_val = jnp.full((nb, 1), -jnp.inf, jnp.float32)
            best_idx = jnp.zeros((nb, 1), jnp.int32)
    hb = jax.lax.bitcast_convert_type(half_v[0], jnp.uint32)
    hb = (hb + jnp.uint32(0x7FFF) + ((hb >> 16) & jnp.uint32(1))) \
        & jnp.uint32(0xFFFF0000)
    w0_bf = jax.lax.bitcast_convert_type(hb, jnp.float32)
    w1_wins = half_v[1] > w0_bf
    best_idx = jnp.where(w1_wins, half_i[1], half_i[0])

    q = jnp.zeros((nb, d), jnp.float32)
    for kb in range(ksize // KC):
        cchunk = cb[kb * KC:(kb + 1) * KC, :]
        lane = jax.lax.broadcasted_iota(jnp.int32, (nb, KC), 1) + kb * KC
        oh = (best_idx == lane).astype(jnp.bfloat16)
        c1 = cchunk.astype(jnp.bfloat16)
        d1 = cchunk - c1.astype(jnp.float32)
        c2 = d1.astype(jnp.bfloat16)
        c3 = (d1 - c2.astype(jnp.float32)).astype(jnp.bfloat16)
        for cpart in (c1, c2, c3):
            q = q + jax.lax.dot_general(
                oh, cpart, (((1,), (0,)), ((), ())),
                preferred_element_type=jnp.float32)

    rn = r - q
    r_ref[...] = rn
    idx_ref[...] = best_idx.reshape(1, nb, 1)
    loss_ref[...] = jnp.full((1, 1, 128), jnp.sum(rn * rn), jnp.float32)
    qout_ref[...] = x_ref[...] - rn


@functools.partial(jax.jit, static_argnames=())
def kernel(x, codebooks):
    b, t, d = x.shape
    nq, ksize, _ = codebooks.shape
    n = b * t
    nb = n // NBLK
    x2 = x.reshape(n, d)

    grid = (NBLK, nq)
    qout, idx3, lossp = pl.pallas_call(
        functools.partial(_rvq_kernel, nq=nq, ksize=ksize),
        grid=grid,
        in_specs=[
            pl.BlockSpec((nb, d), lambda i, l: (i, 0)),
            pl.BlockSpec((1, ksize, d), lambda i, l: (l, 0, 0)),
            pl.BlockSpec((1, 1, ksize), lambda i, l: (l, 0, 0)),
        ],
        out_specs=[
            pl.BlockSpec((nb, d), lambda i, l: (i, 0)),
            pl.BlockSpec((1, nb, 1), lambda i, l: (l * NBLK + i, 0, 0)),
            pl.BlockSpec((1, 1, 128), lambda i, l: (l * NBLK + i, 0, 0)),
        ],
        out_shape=[
            jax.ShapeDtypeStruct((n, d), jnp.float32),
            jax.ShapeDtypeStruct((nq * NBLK, nb, 1), jnp.int32),
            jax.ShapeDtypeStruct((nq * NBLK, 1, 128), jnp.float32),
        ],
        scratch_shapes=[pltpu.VMEM((nb, d), jnp.float32)],
        compiler_params=pltpu.CompilerParams(
            dimension_semantics=("arbitrary", "arbitrary"),
        ),
    )(x2, codebooks, jnp.sum(codebooks**2, axis=2)[:, None, :])

    indices = idx3.reshape(nq, b, t)
    losses = 1.25 * lossp[:, 0, 0].reshape(nq, NBLK).sum(axis=1) / (n * d)
    return qout.reshape(b, t, d), indices, losses


# speedup vs baseline: 2.9708x; 1.7321x over previous
import functools

import jax
import jax.numpy as jnp
from jax import lax
from jax.experimental import pallas as pl
from jax.experimental.pallas import tpu as pltpu
from jax.experimental.pallas import tpu_sc as plsc

NBLK = 32
KC = 1024

_SC_WORKERS = 32
_SC_CHUNK = 256


def _bf16_round(v):
    hb = jax.lax.bitcast_convert_type(v, jnp.uint32)
    hb = (hb + jnp.uint32(0x7FFF) + ((hb >> 16) & jnp.uint32(1))) \
        & jnp.uint32(0xFFFF0000)
    return jax.lax.bitcast_convert_type(hb, jnp.float32)


def _argmax_mirror(r, cb, cn_row, ksize):
    nb = r.shape[0]
    fn = jnp.sum(r * r, axis=1, keepdims=True)
    rbf = r.astype(jnp.bfloat16)
    nchunks = ksize // KC
    half = nchunks // 2
    half_v = []
    half_i = []
    best_val = jnp.full((nb, 1), -jnp.inf, jnp.float32)
    best_idx = jnp.zeros((nb, 1), jnp.int32)
    for kb in range(nchunks):
        cchunk = cb[kb * KC:(kb + 1) * KC, :]
        cn = cn_row[:, kb * KC:(kb + 1) * KC]
        cbf = cchunk.astype(jnp.bfloat16)
        mm = jax.lax.dot_general(
            rbf, cbf, (((1,), (1,)), ((), ())),
            preferred_element_type=jnp.float32)
        u = -((fn - 2.0 * mm) + cn)
        m = jnp.max(u, axis=1, keepdims=True)
        a = jnp.argmax(u, axis=1).astype(jnp.int32)[:, None] + kb * KC
        upd = m > best_val
        best_val = jnp.where(upd, m, best_val)
        best_idx = jnp.where(upd, a, best_idx)
        if kb == half - 1 or kb == nchunks - 1:
            half_v.append(best_val)
            half_i.append(best_idx)
            best_val = jnp.full((nb, 1), -jnp.inf, jnp.float32)
            best_idx = jnp.zeros((nb, 1), jnp.int32)
    w1_wins = half_v[1] > _bf16_round(half_v[0])
    return jnp.where(w1_wins, half_i[1], half_i[0])


def _step_kernel(r_ref, q_ref, cb_ref, cn_ref, idx_ref, rout_ref, loss_ref, *,
                 ksize):
    d = r_ref.shape[1]
    r = r_ref[...] - q_ref[:, 0:d]
    rout_ref[...] = r
    loss_ref[...] = jnp.full((1, 1, 128), jnp.sum(r * r), jnp.float32)
    best_idx = _argmax_mirror(r, cb_ref[0], cn_ref[0], ksize)
    idx_ref[...] = best_idx.reshape(1, r.shape[0], 1)


def _final_kernel(x_ref, r_ref, q_ref, qout_ref, loss_ref):
    rn = r_ref[...] - q_ref[:, 0:r_ref.shape[1]]
    loss_ref[...] = jnp.full((1, 1, 128), jnp.sum(rn * rn), jnp.float32)
    qout_ref[...] = x_ref[...] - rn


def _tc_step(r, q, cb, cn, nblk):
    n, d = r.shape
    ksize = cb.shape[0]
    nb = n // nblk
    idx3, rout, lossp = pl.pallas_call(
        functools.partial(_step_kernel, ksize=ksize),
        grid=(nblk,),
        in_specs=[
            pl.BlockSpec((nb, d), lambda i: (i, 0)),
            pl.BlockSpec((nb, 128), lambda i: (i, 0)),
            pl.BlockSpec((1, ksize, d), lambda i: (0, 0, 0)),
            pl.BlockSpec((1, 1, ksize), lambda i: (0, 0, 0)),
        ],
        out_specs=[
            pl.BlockSpec((1, nb, 1), lambda i: (i, 0, 0)),
            pl.BlockSpec((nb, d), lambda i: (i, 0)),
            pl.BlockSpec((1, 1, 128), lambda i: (i, 0, 0)),
        ],
        out_shape=[
            jax.ShapeDtypeStruct((nblk, nb, 1), jnp.int32),
            jax.ShapeDtypeStruct((n, d), jnp.float32),
            jax.ShapeDtypeStruct((nblk, 1, 128), jnp.float32),
        ],
        compiler_params=pltpu.CompilerParams(
            dimension_semantics=("arbitrary",),
        ),
    )(r, q, cb[None], cn[None, None])
    return idx3.reshape(n), rout, jnp.sum(lossp[:, 0, 0])


def _tc_final(x2, r, q, nblk):
    n, d = x2.shape
    nb = n // nblk
    qout, lossp = pl.pallas_call(
        _final_kernel,
        grid=(nblk,),
        in_specs=[
            pl.BlockSpec((nb, d), lambda i: (i, 0)),
            pl.BlockSpec((nb, d), lambda i: (i, 0)),
            pl.BlockSpec((nb, 128), lambda i: (i, 0)),
        ],
        out_specs=[
            pl.BlockSpec((nb, d), lambda i: (i, 0)),
            pl.BlockSpec((1, 1, 128), lambda i: (i, 0, 0)),
        ],
        out_shape=[
            jax.ShapeDtypeStruct((n, d), jnp.float32),
            jax.ShapeDtypeStruct((nblk, 1, 128), jnp.float32),
        ],
        compiler_params=pltpu.CompilerParams(
            dimension_semantics=("arbitrary",),
        ),
    )(x2, r, q)
    return qout, jnp.sum(lossp[:, 0, 0])


def _make_sc_gather(v, d, b):
    b_per_w = b // _SC_WORKERS
    nchunks = b_per_w // _SC_CHUNK
    mesh = plsc.VectorSubcoreMesh(core_axis_name="c", subcore_axis_name="s")

    @functools.partial(
        pl.kernel, mesh=mesh,
        out_type=jax.ShapeDtypeStruct((b, d), jnp.float32),
        scratch_types=[
            pltpu.VMEM((_SC_CHUNK,), jnp.int32),
            pltpu.VMEM((_SC_CHUNK, d), jnp.float32),
            pltpu.SemaphoreType.DMA,
        ],
    )
    def k(table_hbm, idx_hbm, out_hbm, idx_v, rows_v, sem):
        wid = lax.axis_index("s") * 2 + lax.axis_index("c")
        base = wid * b_per_w
        for c in range(nchunks):
            off = base + c * _SC_CHUNK
            pltpu.sync_copy(idx_hbm.at[pl.ds(off, _SC_CHUNK)], idx_v)
            pltpu.async_copy(table_hbm.at[idx_v], rows_v, sem).wait()
            pltpu.sync_copy(rows_v, out_hbm.at[pl.ds(off, _SC_CHUNK)])

    return k


@jax.jit
def kernel(x, codebooks):
    b, t, d = x.shape
    nq, ksize, _ = codebooks.shape
    n = b * t
    x2 = x.reshape(n, d)
    cns = jnp.sum(codebooks**2, axis=2)

    cb_pad = jnp.pad(codebooks, ((0, 0), (0, 0), (0, 128 - d)))
    sc_gather = _make_sc_gather(ksize, 128, n)
    r = x2
    q = jnp.zeros((n, 128), jnp.float32)
    loss_parts = []
    indices = []
    for i in range(nq):
        idx, r, lp = _tc_step(r, q, codebooks[i], cns[i], NBLK)
        if i > 0:
            loss_parts.append(lp)
        indices.append(idx)
        q = sc_gather(cb_pad[i], idx)
    qout, lp_last = _tc_final(x2, r, q, NBLK)
    loss_parts.append(lp_last)

    losses = 1.25 * jnp.stack(loss_parts) / (n * d)
    return (qout.reshape(b, t, d),
            jnp.stack(indices).reshape(nq, b, t),
            losses)


# speedup vs baseline: 4.5954x; 1.5469x over previous
import functools

import jax
import jax.numpy as jnp
from jax import lax
from jax.experimental import pallas as pl
from jax.experimental.pallas import tpu as pltpu
from jax.experimental.pallas import tpu_sc as plsc

NBLK = 32
KC = 1024

_SC_WORKERS = 32
_SC_CHUNK = 256


def _bf16_round(v):
    hb = jax.lax.bitcast_convert_type(v, jnp.uint32)
    hb = (hb + jnp.uint32(0x7FFF) + ((hb >> 16) & jnp.uint32(1))) \
        & jnp.uint32(0xFFFF0000)
    return jax.lax.bitcast_convert_type(hb, jnp.float32)


def _step_kernel(r_ref, q_ref, cb2_ref, cn_ref, idx_ref, rout_ref, loss_ref,
                 dscr, *, ksize):
    d = r_ref.shape[1]
    nb = r_ref.shape[0]
    r = r_ref[...] - q_ref[:, 0:d]
    rout_ref[...] = r
    fn = jnp.sum(r * r, axis=1, keepdims=True)
    loss_ref[...] = jnp.full((1, 1, 128), jnp.sum(fn), jnp.float32)
    rbf = r.astype(jnp.bfloat16)
    nchunks = ksize // KC
    half = nchunks // 2
    win_min = []
    win_idx = []
    big = jnp.int32(ksize)
    for w in range(2):
        wmin = jnp.full((nb, 1), jnp.inf, jnp.float32)
        for c in range(half):
            kb = w * half + c
            cchunk = cb2_ref[0, kb * KC:(kb + 1) * KC, :]
            cn = cn_ref[0, :, kb * KC:(kb + 1) * KC]
            mm2 = jax.lax.dot_general(
                rbf, cchunk, (((1,), (1,)), ((), ())),
                preferred_element_type=jnp.float32)
            dv = (fn - mm2) + cn
            dscr[:, c * KC:(c + 1) * KC] = dv
            wmin = jnp.minimum(wmin, jnp.min(dv, axis=1, keepdims=True))
        idxw = jnp.full((nb, 1), big, jnp.int32)
        for c in range(half):
            kb = w * half + c
            dv = dscr[:, c * KC:(c + 1) * KC]
            io = jax.lax.broadcasted_iota(jnp.int32, (nb, KC), 1) + kb * KC
            cand = jnp.where(dv <= wmin, io, big)
            idxw = jnp.minimum(idxw, jnp.min(cand, axis=1, keepdims=True))
        win_min.append(wmin)
        win_idx.append(idxw)
    w1_wins = win_min[1] < _bf16_round(win_min[0])
    best_idx = jnp.where(w1_wins, win_idx[1], win_idx[0])
    idx_ref[...] = best_idx.reshape(1, nb, 1)


def _final_kernel(x_ref, r_ref, q_ref, qout_ref, loss_ref):
    rn = r_ref[...] - q_ref[:, 0:r_ref.shape[1]]
    loss_ref[...] = jnp.full((1, 1, 128), jnp.sum(rn * rn), jnp.float32)
    qout_ref[...] = x_ref[...] - rn


def _tc_step(r, q, cb2, cn, nblk):
    n, d = r.shape
    ksize = cb2.shape[0]
    nb = n // nblk
    idx3, rout, lossp = pl.pallas_call(
        functools.partial(_step_kernel, ksize=ksize),
        grid=(nblk,),
        in_specs=[
            pl.BlockSpec((nb, d), lambda i: (i, 0)),
            pl.BlockSpec((nb, 128), lambda i: (i, 0)),
            pl.BlockSpec((1, ksize, d), lambda i: (0, 0, 0)),
            pl.BlockSpec((1, 1, ksize), lambda i: (0, 0, 0)),
        ],
        out_specs=[
            pl.BlockSpec((1, nb, 1), lambda i: (i, 0, 0)),
            pl.BlockSpec((nb, d), lambda i: (i, 0)),
            pl.BlockSpec((1, 1, 128), lambda i: (i, 0, 0)),
        ],
        out_shape=[
            jax.ShapeDtypeStruct((nblk, nb, 1), jnp.int32),
            jax.ShapeDtypeStruct((n, d), jnp.float32),
            jax.ShapeDtypeStruct((nblk, 1, 128), jnp.float32),
        ],
        scratch_shapes=[pltpu.VMEM((nb, ksize // 2), jnp.float32)],
        compiler_params=pltpu.CompilerParams(
            dimension_semantics=("arbitrary",),
        ),
    )(r, q, cb2[None], cn[None, None])
    return idx3.reshape(n), rout, jnp.sum(lossp[:, 0, 0])


def _tc_final(x2, r, q, nblk):
    n, d = x2.shape
    nb = n // nblk
    qout, lossp = pl.pallas_call(
        _final_kernel,
        grid=(nblk,),
        in_specs=[
            pl.BlockSpec((nb, d), lambda i: (i, 0)),
            pl.BlockSpec((nb, d), lambda i: (i, 0)),
            pl.BlockSpec((nb, 128), lambda i: (i, 0)),
        ],
        out_specs=[
            pl.BlockSpec((nb, d), lambda i: (i, 0)),
            pl.BlockSpec((1, 1, 128), lambda i: (i, 0, 0)),
        ],
        out_shape=[
            jax.ShapeDtypeStruct((n, d), jnp.float32),
            jax.ShapeDtypeStruct((nblk, 1, 128), jnp.float32),
        ],
        compiler_params=pltpu.CompilerParams(
            dimension_semantics=("arbitrary",),
        ),
    )(x2, r, q)
    return qout, jnp.sum(lossp[:, 0, 0])


def _make_sc_gather(v, d, b):
    b_per_w = b // _SC_WORKERS
    nchunks = b_per_w // _SC_CHUNK
    mesh = plsc.VectorSubcoreMesh(core_axis_name="c", subcore_axis_name="s")

    @functools.partial(
        pl.kernel, mesh=mesh,
        out_type=jax.ShapeDtypeStruct((b, d), jnp.float32),
        scratch_types=[
            pltpu.VMEM((_SC_CHUNK,), jnp.int32),
            pltpu.VMEM((_SC_CHUNK, d), jnp.float32),
            pltpu.SemaphoreType.DMA,
        ],
    )
    def k(table_hbm, idx_hbm, out_hbm, idx_v, rows_v, sem):
        wid = lax.axis_index("s") * 2 + lax.axis_index("c")
        base = wid * b_per_w
        for c in range(nchunks):
            off = base + c * _SC_CHUNK
            pltpu.sync_copy(idx_hbm.at[pl.ds(off, _SC_CHUNK)], idx_v)
            pltpu.async_copy(table_hbm.at[idx_v], rows_v, sem).wait()
            pltpu.sync_copy(rows_v, out_hbm.at[pl.ds(off, _SC_CHUNK)])

    return k


@jax.jit
def kernel(x, codebooks):
    b, t, d = x.shape
    nq, ksize, _ = codebooks.shape
    n = b * t
    x2 = x.reshape(n, d)
    cns = jnp.sum(codebooks**2, axis=2)
    cb2bf = (codebooks * 2.0).astype(jnp.bfloat16)

    cb_pad = jnp.pad(codebooks, ((0, 0), (0, 0), (0, 128 - d)))
    sc_gather = _make_sc_gather(ksize, 128, n)
    r = x2
    q = jnp.zeros((n, 128), jnp.float32)
    loss_parts = []
    indices = []
    for i in range(nq):
        idx, r, lp = _tc_step(r, q, cb2bf[i], cns[i], NBLK)
        if i > 0:
            loss_parts.append(lp)
        indices.append(idx)
        q = sc_gather(cb_pad[i], idx)
    qout, lp_last = _tc_final(x2, r, q, NBLK)
    loss_parts.append(lp_last)

    losses = 1.25 * jnp.stack(loss_parts) / (n * d)
    return (qout.reshape(b, t, d),
            jnp.stack(indices).reshape(nq, b, t),
            losses)


# speedup vs baseline: 5.6135x; 1.2216x over previous
import functools

import jax
import jax.numpy as jnp
from jax import lax
from jax.experimental import pallas as pl
from jax.experimental.pallas import tpu as pltpu
from jax.experimental.pallas import tpu_sc as plsc

NBLK = 32
KC = 1024

_SC_WORKERS = 32
_SC_CHUNK = 256


def _bf16_round(v):
    hb = jax.lax.bitcast_convert_type(v, jnp.uint32)
    hb = (hb + jnp.uint32(0x7FFF) + ((hb >> 16) & jnp.uint32(1))) \
        & jnp.uint32(0xFFFF0000)
    return jax.lax.bitcast_convert_type(hb, jnp.float32)


def _step_kernel(r_ref, q_ref, cb2_ref, cn_ref, idx_ref, rout_ref, loss_ref,
                 dscr, *, ksize):
    d = r_ref.shape[1]
    nb = r_ref.shape[0]
    r = r_ref[...] - q_ref[:, 0:d]
    rout_ref[...] = r
    fn = jnp.sum(r * r, axis=1, keepdims=True)
    loss_ref[...] = jnp.full((1, 1, 128), jnp.sum(fn), jnp.float32)
    rbf = r.astype(jnp.bfloat16)
    nchunks = ksize // KC
    half = nchunks // 2
    win_min = []
    win_idx = []
    bigf = jnp.float32(ksize)
    io0 = jax.lax.broadcasted_iota(jnp.int32, (nb, KC), 1).astype(jnp.float32)
    for w in range(2):
        wmin = jnp.full((nb, 1), jnp.inf, jnp.float32)
        for c in range(half):
            kb = w * half + c
            cchunk = cb2_ref[0, kb * KC:(kb + 1) * KC, :]
            cn = cn_ref[0, :, kb * KC:(kb + 1) * KC]
            mm2 = jax.lax.dot_general(
                rbf, cchunk, (((1,), (1,)), ((), ())),
                preferred_element_type=jnp.float32)
            dv = (fn - mm2) + cn
            dscr[:, c * KC:(c + 1) * KC] = dv
            wmin = jnp.minimum(wmin, jnp.min(dv, axis=1, keepdims=True))
        idxw = jnp.full((nb, 1), bigf, jnp.float32)
        for c in range(half):
            kb = w * half + c
            dv = dscr[:, c * KC:(c + 1) * KC]
            cand = jnp.where(dv <= wmin, io0, bigf)
            idxc = jnp.min(cand, axis=1, keepdims=True) + jnp.float32(kb * KC)
            idxw = jnp.minimum(idxw, idxc)
        win_min.append(wmin)
        win_idx.append(idxw)
    w1_wins = win_min[1] < _bf16_round(win_min[0])
    best_idx = jnp.where(w1_wins, win_idx[1], win_idx[0]).astype(jnp.int32)
    idx_ref[...] = best_idx.reshape(1, nb, 1)


def _final_kernel(x_ref, r_ref, q_ref, qout_ref, loss_ref):
    rn = r_ref[...] - q_ref[:, 0:r_ref.shape[1]]
    loss_ref[...] = jnp.full((1, 1, 128), jnp.sum(rn * rn), jnp.float32)
    qout_ref[...] = x_ref[...] - rn


def _tc_step(r, q, cb2, cn, nblk):
    n, d = r.shape
    ksize = cb2.shape[0]
    nb = n // nblk
    idx3, rout, lossp = pl.pallas_call(
        functools.partial(_step_kernel, ksize=ksize),
        grid=(nblk,),
        in_specs=[
            pl.BlockSpec((nb, d), lambda i: (i, 0)),
            pl.BlockSpec((nb, 128), lambda i: (i, 0)),
            pl.BlockSpec((1, ksize, d), lambda i: (0, 0, 0)),
            pl.BlockSpec((1, 1, ksize), lambda i: (0, 0, 0)),
        ],
        out_specs=[
            pl.BlockSpec((1, nb, 1), lambda i: (i, 0, 0)),
            pl.BlockSpec((nb, d), lambda i: (i, 0)),
            pl.BlockSpec((1, 1, 128), lambda i: (i, 0, 0)),
        ],
        out_shape=[
            jax.ShapeDtypeStruct((nblk, nb, 1), jnp.int32),
            jax.ShapeDtypeStruct((n, d), jnp.float32),
            jax.ShapeDtypeStruct((nblk, 1, 128), jnp.float32),
        ],
        scratch_shapes=[pltpu.VMEM((nb, ksize // 2), jnp.float32)],
        compiler_params=pltpu.CompilerParams(
            dimension_semantics=("arbitrary",),
        ),
    )(r, q, cb2[None], cn[None, None])
    return idx3.reshape(n), rout, jnp.sum(lossp[:, 0, 0])


def _tc_final(x2, r, q, nblk):
    n, d = x2.shape
    nb = n // nblk
    qout, lossp = pl.pallas_call(
        _final_kernel,
        grid=(nblk,),
        in_specs=[
            pl.BlockSpec((nb, d), lambda i: (i, 0)),
            pl.BlockSpec((nb, d), lambda i: (i, 0)),
            pl.BlockSpec((nb, 128), lambda i: (i, 0)),
        ],
        out_specs=[
            pl.BlockSpec((nb, d), lambda i: (i, 0)),
            pl.BlockSpec((1, 1, 128), lambda i: (i, 0, 0)),
        ],
        out_shape=[
            jax.ShapeDtypeStruct((n, d), jnp.float32),
            jax.ShapeDtypeStruct((nblk, 1, 128), jnp.float32),
        ],
        compiler_params=pltpu.CompilerParams(
            dimension_semantics=("arbitrary",),
        ),
    )(x2, r, q)
    return qout, jnp.sum(lossp[:, 0, 0])


def _make_sc_gather(v, d, b):
    b_per_w = b // _SC_WORKERS
    nchunks = b_per_w // _SC_CHUNK
    mesh = plsc.VectorSubcoreMesh(core_axis_name="c", subcore_axis_name="s")

    @functools.partial(
        pl.kernel, mesh=mesh,
        out_type=jax.ShapeDtypeStruct((b, d), jnp.float32),
        scratch_types=[
            pltpu.VMEM((_SC_CHUNK,), jnp.int32),
            pltpu.VMEM((_SC_CHUNK, d), jnp.float32),
            pltpu.SemaphoreType.DMA,
        ],
    )
    def k(table_hbm, idx_hbm, out_hbm, idx_v, rows_v, sem):
        wid = lax.axis_index("s") * 2 + lax.axis_index("c")
        base = wid * b_per_w
        for c in range(nchunks):
            off = base + c * _SC_CHUNK
            pltpu.sync_copy(idx_hbm.at[pl.ds(off, _SC_CHUNK)], idx_v)
            pltpu.async_copy(table_hbm.at[idx_v], rows_v, sem).wait()
            pltpu.sync_copy(rows_v, out_hbm.at[pl.ds(off, _SC_CHUNK)])

    return k


@jax.jit
def kernel(x, codebooks):
    b, t, d = x.shape
    nq, ksize, _ = codebooks.shape
    n = b * t
    x2 = x.reshape(n, d)
    cns = jnp.sum(codebooks**2, axis=2)
    cb2bf = (codebooks * 2.0).astype(jnp.bfloat16)

    cb_pad = jnp.pad(codebooks, ((0, 0), (0, 0), (0, 128 - d)))
    sc_gather = _make_sc_gather(ksize, 128, n)
    r = x2
    q = jnp.zeros((n, 128), jnp.float32)
    loss_parts = []
    indices = []
    for i in range(nq):
        idx, r, lp = _tc_step(r, q, cb2bf[i], cns[i], NBLK)
        if i > 0:
            loss_parts.append(lp)
        indices.append(idx)
        q = sc_gather(cb_pad[i], idx)
    qout, lp_last = _tc_final(x2, r, q, NBLK)
    loss_parts.append(lp_last)

    losses = 1.25 * jnp.stack(loss_parts) / (n * d)
    return (qout.reshape(b, t, d),
            jnp.stack(indices).reshape(nq, b, t),
            losses)
